# baseline (speedup 1.0000x reference)
import dataclasses

import jax
import jax.numpy as jnp
from jax import lax
from jax.experimental import pallas as pl
from jax.experimental.layout import Layout, with_layout_constraint
from jax.experimental.pallas import tpu as pltpu
from jax.experimental.pallas import tpu_sc as plsc

DIM = 64
NUM_TILES = 32
CHUNK = 256
LANES = 16


def kernel(x, table):
    batch, seq = x.shape
    num_idx = batch * seq
    chunks_per_seq = batch // CHUNK
    num_chunks = num_idx // CHUNK
    chunks_per_tile = num_chunks // NUM_TILES
    idx = jnp.transpose(x).reshape(num_idx)

    table = with_layout_constraint(
        table, Layout(major_to_minor=(0, 1), tiling=((16,),)))

    mesh = plsc.VectorSubcoreMesh(core_axis_name="c", subcore_axis_name="s")

    cp = pltpu.CompilerParams()
    if "needs_layout_passes" in pltpu.CompilerParams.__dataclass_fields__:
        cp = dataclasses.replace(cp, needs_layout_passes=False)

    @pl.kernel(
        out_type=jax.ShapeDtypeStruct((seq, DIM, batch), table.dtype),
        mesh=mesh,
        compiler_params=cp,
        scratch_types=[
            pltpu.VMEM((CHUNK,), jnp.int32),
            pltpu.VMEM((CHUNK,), jnp.int32),
            pltpu.VMEM((CHUNK, DIM), jnp.float32),
            pltpu.VMEM((CHUNK, DIM), jnp.float32),
            pltpu.VMEM((DIM, CHUNK), jnp.float32),
            pltpu.VMEM((DIM, CHUNK), jnp.float32),
            pltpu.SemaphoreType.DMA,
            pltpu.SemaphoreType.DMA,
            pltpu.SemaphoreType.DMA,
            pltpu.SemaphoreType.DMA,
        ],
    )
    def gather_kernel(table_hbm, idx_hbm, out_hbm,
                      idx_v0, idx_v1, rows_v0, rows_v1, out_t0, out_t1,
                      gsem0, gsem1, wsem0, wsem1):
        wid = lax.axis_index("s") * 2 + lax.axis_index("c")
        g0 = wid * chunks_per_tile
        iota = lax.iota(jnp.int32, LANES)

        def issue_gather(g, idx_v, rows_v, gsem):
            s = g // chunks_per_seq
            b0 = (g % chunks_per_seq) * CHUNK
            pltpu.sync_copy(idx_hbm.at[pl.ds(s * batch + b0, CHUNK)], idx_v)
            pltpu.async_copy(table_hbm.at[idx_v], rows_v, gsem)

        def out_slice(g):
            s = g // chunks_per_seq
            b0 = (g % chunks_per_seq) * CHUNK
            return out_hbm.at[s, :, pl.ds(b0, CHUNK)]

        def process(j, idx_cur, rows_cur, out_cur, gsem_cur, wsem_cur,
                    idx_nxt, rows_nxt, gsem_nxt):
            g = g0 + j

            @pl.when(j + 1 < chunks_per_tile)
            def _():
                issue_gather(g + 1, idx_nxt, rows_nxt, gsem_nxt)

            pltpu.make_async_copy(table_hbm.at[idx_cur], rows_cur,
                                  gsem_cur).wait()

            @pl.when(j >= 2)
            def _():
                pltpu.make_async_copy(out_cur, out_slice(g - 2),
                                      wsem_cur).wait()

            @pl.loop(0, DIM)
            def _(c):
                c_vec = iota * 0 + c
                for k in range(CHUNK // LANES):
                    v = plsc.load_gather(rows_cur,
                                         [k * LANES + iota, c_vec])
                    out_cur[c, pl.ds(k * LANES, LANES)] = v

            pltpu.async_copy(out_cur, out_slice(g), wsem_cur)

        issue_gather(g0, idx_v0, rows_v0, gsem0)

        @pl.loop(0, chunks_per_tile // 2)
        def _(i):
            process(2 * i, idx_v0, rows_v0, out_t0, gsem0, wsem0,
                    idx_v1, rows_v1, gsem1)
            process(2 * i + 1, idx_v1, rows_v1, out_t1, gsem1, wsem1,
                    idx_v0, rows_v0, gsem0)

        last = g0 + chunks_per_tile
        pltpu.make_async_copy(out_t0, out_slice(last - 2), wsem0).wait()
        pltpu.make_async_copy(out_t1, out_slice(last - 1), wsem1).wait()

    out = gather_kernel(table, idx)
    return jnp.transpose(out, (2, 0, 1))


# speedup vs baseline: 1.3700x; 1.3700x over previous
import jax
import jax.numpy as jnp
from jax import lax
from jax.experimental import pallas as pl
from jax.experimental.layout import Layout, with_layout_constraint
from jax.experimental.pallas import tpu as pltpu
from jax.experimental.pallas import tpu_sc as plsc

DIM = 64
NUM_TILES = 32
CHUNK = 200
NBUF = 4


def kernel(x, table):
    batch, seq = x.shape
    num_idx = batch * seq
    per_tile = num_idx // NUM_TILES
    steps = per_tile // CHUNK
    idx = x.reshape(num_idx)

    table = with_layout_constraint(
        table, Layout(major_to_minor=(0, 1), tiling=((16,),)))

    mesh = plsc.VectorSubcoreMesh(core_axis_name="c", subcore_axis_name="s")

    @pl.kernel(
        out_type=jax.ShapeDtypeStruct((num_idx, DIM), table.dtype),
        mesh=mesh,
        scratch_types=(
            [pltpu.VMEM((CHUNK,), jnp.int32) for _ in range(NBUF)]
            + [pltpu.VMEM((CHUNK, DIM), jnp.float32) for _ in range(NBUF)]
            + [pltpu.SemaphoreType.DMA for _ in range(NBUF)]
        ),
    )
    def gather_kernel(table_hbm, idx_hbm, out_hbm, *scratch):
        idx_vs = scratch[:NBUF]
        rows_vs = scratch[NBUF:2 * NBUF]
        gsems = scratch[2 * NBUF:]
        wid = lax.axis_index("s") * 2 + lax.axis_index("c")
        tile_base = wid * per_tile

        def out_slice(j):
            return out_hbm.at[pl.ds(tile_base + j * CHUNK, CHUNK)]

        @pl.loop(0, steps // NBUF)
        def _(i):
            j0 = i * NBUF
            descs = []
            for k in range(NBUF):
                base = tile_base + (j0 + k) * CHUNK
                pltpu.sync_copy(idx_hbm.at[pl.ds(base, CHUNK)], idx_vs[k])
                descs.append(pltpu.async_copy(
                    table_hbm.at[idx_vs[k]], rows_vs[k], gsems[k]))
            for k in range(NBUF):
                descs[k].wait()
                pltpu.sync_copy(rows_vs[k], out_slice(j0 + k))

    out = gather_kernel(table, idx)
    return out.reshape(batch, seq, DIM)
